# baseline (speedup 1.0000x reference)
import functools

import jax
import jax.numpy as jnp
from jax import lax
from jax.experimental import pallas as pl
from jax.experimental.pallas import tpu as pltpu
from jax.experimental.pallas import tpu_sc as plsc

_PREC = lax.Precision.HIGHEST
_NC = 2
_NS = 16
_CH = 56
_NSLOT = 6


def _seg_pad(n):
    rows_per_sub = ((n + _NS - 1) // _NS + 7) // 8 * 8
    return rows_per_sub, rows_per_sub * _NS


def _edge_segment_sum(h, src3, dst3, zeros):
    n, d = h.shape
    e = src3.shape[0]
    nw = _NC * _NS
    ch = _CH
    per_w = e // nw
    steps = per_w // ch
    rows_per_sub, npad = _seg_pad(n)
    mesh = plsc.VectorSubcoreMesh(core_axis_name="c", subcore_axis_name="s")

    ds2 = 2 * _NSLOT
    nblk = steps // ds2
    assert steps == ds2 * nblk
    last = steps - 1

    @functools.partial(
        pl.kernel,
        out_type=jax.ShapeDtypeStruct((_NC, npad, d), jnp.float32),
        mesh=mesh,
        scratch_types=[
            [pltpu.VMEM((ch,), jnp.int32)] * _NSLOT,
            [pltpu.VMEM((ch,), jnp.int32)] * (2 * _NSLOT),
            [pltpu.VMEM((ch, d), jnp.float32)] * _NSLOT,
            pltpu.VMEM_SHARED((npad, d), jnp.float32),
            [pltpu.SemaphoreType.DMA] * _NSLOT,
            [pltpu.SemaphoreType.DMA] * _NSLOT,
            [pltpu.SemaphoreType.DMA] * _NSLOT,
            [pltpu.SemaphoreType.DMA] * _NSLOT,
        ],
    )
    def k(h_hbm, src_hbm, dst_hbm, z_hbm, out_hbm,
          sidx, didx, rows, agg, gsem, ssem, isem, dsem):
        c = lax.axis_index("c")
        s = lax.axis_index("s")
        wid = s * _NC + c
        base = wid * per_w
        rbase = s * rows_per_sub

        def soff(i):
            return base + jnp.minimum(i, last) * ch

        def idx_cp(i, q, dslot):
            a = pltpu.make_async_copy(src_hbm.at[pl.ds(soff(i), ch)],
                                      sidx[q], isem[q])
            b = pltpu.make_async_copy(dst_hbm.at[pl.ds(soff(i), ch)],
                                      didx[dslot], dsem[q])
            return a, b

        def gath(q):
            return pltpu.make_async_copy(h_hbm.at[sidx[q]], rows[q], gsem[q])

        def scat(q, dslot):
            return pltpu.make_async_copy(rows[q], agg.at[didx[dslot]],
                                         ssem[q])

        pltpu.sync_copy(z_hbm.at[pl.ds(rbase, rows_per_sub)],
                        agg.at[pl.ds(rbase, rows_per_sub)])
        plsc.subcore_barrier()

        def halfblock(c0, lo, hi, first):
            gs = []
            for q in range(_NSLOT):
                if not first:
                    scat(q, (hi + q) % ds2).wait()
                a, b = idx_cp(c0 + lo + q, q, lo + q)
                a.wait()
                b.wait()
                g = gath(q)
                g.start()
                gs.append(g)
            for q in range(_NSLOT):
                gs[q].wait()
                scat(q, lo + q).start(add=True)
                for cp in idx_cp(c0 + hi + q, q, (hi + q) % ds2):
                    cp.start()

        def block(j, first):
            c0 = ds2 * j
            if first:
                for q in range(_NSLOT):
                    for cp in idx_cp(c0 + q, q, q):
                        cp.start()
            halfblock(c0, 0, _NSLOT, first)
            halfblock(c0, _NSLOT, ds2, False)

        block(0, True)

        @pl.loop(1, nblk)
        def _(j):
            block(j, False)

        for q in range(_NSLOT):
            scat(q, _NSLOT + q).wait()
            a, b = idx_cp(0, q, q)
            a.wait()
            b.wait()

        plsc.subcore_barrier()
        pltpu.sync_copy(agg.at[pl.ds(rbase, rows_per_sub)],
                        out_hbm.at[c].at[pl.ds(rbase, rows_per_sub)])

    return k(h, src3, dst3, zeros)


def _pad_edges(src, dst, n):
    e = src.shape[0]
    nw = _NC * _NS
    per_w = -(-e // (nw * 2 * _NSLOT * _CH)) * 2 * _NSLOT * _CH
    epad = nw * per_w
    _, npad = _seg_pad(n)
    pad_ar = jnp.arange(epad - e, dtype=jnp.int32)
    trash = n + pad_ar % (npad - n)
    src_p = jnp.concatenate([src, pad_ar % n])
    dst_p = jnp.concatenate([dst, trash])
    return src_p, dst_p


def _gin_layer(h, parts, n2g_col, eps, w1, b1, g1, be1, w2, b2, gl, bel,
               pool_input):
    n, d = h.shape
    hh = w1.shape[1]
    ng = 64

    def body(*refs):
        if pool_input:
            (h_ref, p_ref, n2g_ref, w1_ref, b1_ref, g1_ref, be1_ref,
             w2_ref, b2_ref, gl_ref, bel_ref, eps_ref,
             ho_ref, po_ref, pi_ref) = refs
        else:
            (h_ref, p_ref, n2g_ref, w1_ref, b1_ref, g1_ref, be1_ref,
             w2_ref, b2_ref, gl_ref, bel_ref, eps_ref,
             ho_ref, po_ref) = refs
        hcur = h_ref[...]
        n_rows = hcur.shape[0]
        z = (1.0 + eps_ref[0]) * hcur + p_ref[0, :n_rows] + p_ref[1, :n_rows]
        t = jnp.dot(z, w1_ref[...], precision=_PREC) + b1_ref[...]
        m = jnp.mean(t, axis=0, keepdims=True)
        v = jnp.mean((t - m) ** 2, axis=0, keepdims=True)
        u = jnp.maximum(
            g1_ref[...] * (t - m) / jnp.sqrt(v + 1e-5) + be1_ref[...], 0.0)
        t2 = jnp.dot(u, w2_ref[...], precision=_PREC) + b2_ref[...]
        m2 = jnp.mean(t2, axis=0, keepdims=True)
        v2 = jnp.mean((t2 - m2) ** 2, axis=0, keepdims=True)
        hn = jnp.maximum(
            gl_ref[...] * (t2 - m2) / jnp.sqrt(v2 + 1e-5) + bel_ref[...], 0.0)
        ho_ref[...] = hn
        onehot = (n2g_ref[...] ==
                  lax.broadcasted_iota(jnp.int32, (n, ng), 1)).astype(jnp.float32)
        dn = (((0,), (0,)), ((), ()))
        po_ref[...] = lax.dot_general(onehot, hn, dn, precision=_PREC)
        if pool_input:
            pi_ref[...] = lax.dot_general(onehot, hcur, dn, precision=_PREC)

    out_shapes = [jax.ShapeDtypeStruct((n, hh), jnp.float32),
                  jax.ShapeDtypeStruct((ng, hh), jnp.float32)]
    if pool_input:
        out_shapes.append(jax.ShapeDtypeStruct((ng, d), jnp.float32))
    in_specs = [pl.BlockSpec()] * 11 + [pl.BlockSpec(memory_space=pltpu.SMEM)]
    return pl.pallas_call(
        body,
        out_shape=out_shapes,
        in_specs=in_specs,
        out_specs=[pl.BlockSpec()] * len(out_shapes),
    )(h, parts, n2g_col, w1, b1, g1, be1, w2, b2, gl, bel, eps)


def _readout(pooled, wp, bp):
    ngr, nl, _, hh = pooled.shape
    o = wp.shape[2]

    def body(p_ref, w_ref, b_ref, o_ref):
        for g in range(ngr):
            acc = jnp.zeros((64, o), jnp.float32)
            for l in range(nl):
                acc = acc + jnp.dot(p_ref[g, l], w_ref[l], precision=_PREC)
                acc = acc + b_ref[l]
            nrm = jnp.sqrt(jnp.sum(acc * acc, axis=-1, keepdims=True))
            acc = acc / jnp.maximum(nrm, 1e-5)
            o_ref[:, g * o:(g + 1) * o] = acc

    return pl.pallas_call(
        body,
        out_shape=jax.ShapeDtypeStruct((64, ngr * o), jnp.float32),
    )(pooled, wp, bp)


def kernel(feat0, edge_index0, node2graph0, feat1, edge_index1, node2graph1,
           params):
    n, d = feat0.shape
    lps = [params['layer%d' % l] for l in range(3)]

    _, npad = _seg_pad(n)
    zeros = jnp.zeros((npad, d), jnp.float32)

    def run_graph(feat, edge_index, node2graph):
        src3, dst3 = _pad_edges(edge_index[0], edge_index[1], n)
        n2g_col = node2graph.reshape(n, 1)
        pooled = []
        h = feat
        for l, p in enumerate(lps):
            parts = _edge_segment_sum(h, src3, dst3, zeros)
            eps = jnp.reshape(p['eps'], (1,)).astype(jnp.float32)
            outs = _gin_layer(
                h, parts, n2g_col, eps,
                p['W1'], p['b1'].reshape(1, -1), p['g1'].reshape(1, -1),
                p['be1'].reshape(1, -1), p['W2'], p['b2'].reshape(1, -1),
                p['gL'].reshape(1, -1), p['beL'].reshape(1, -1),
                pool_input=(l == 0))
            if l == 0:
                h, pool_next, pool_in = outs
                pooled.append(pool_in)
            else:
                h, pool_next = outs
            pooled.append(pool_next)
        return jnp.stack(pooled)

    pooled0 = run_graph(feat0, edge_index0, node2graph0)
    pooled1 = run_graph(feat1, edge_index1, node2graph1)
    pooled = jnp.stack([pooled0, pooled1])
    wp = jnp.stack([params['pred%d' % l]['W'] for l in range(4)])
    bp = jnp.stack([params['pred%d' % l]['b'].reshape(1, -1)
                    for l in range(4)])
    return _readout(pooled, wp, bp)


# speedup vs baseline: 1.0415x; 1.0415x over previous
import functools

import jax
import jax.numpy as jnp
from jax import lax
from jax.experimental import pallas as pl
from jax.experimental.pallas import tpu as pltpu
from jax.experimental.pallas import tpu_sc as plsc

_PREC = lax.Precision.HIGHEST
_NC = 2
_NS = 16
_CH = 72
_NSLOT = 5


def _seg_pad(n):
    rows_per_sub = ((n + _NS - 1) // _NS + 7) // 8 * 8
    return rows_per_sub, rows_per_sub * _NS


def _edge_segment_sum(h, src3, dst3, zeros):
    n, d = h.shape
    e = src3.shape[0]
    nw = _NC * _NS
    ch = _CH
    per_w = e // nw
    steps = per_w // ch
    rows_per_sub, npad = _seg_pad(n)
    mesh = plsc.VectorSubcoreMesh(core_axis_name="c", subcore_axis_name="s")

    ds2 = 2 * _NSLOT
    nblk = steps // ds2
    assert steps == ds2 * nblk
    last = steps - 1

    @functools.partial(
        pl.kernel,
        out_type=jax.ShapeDtypeStruct((_NC, npad, d), jnp.float32),
        mesh=mesh,
        scratch_types=[
            [pltpu.VMEM((ch,), jnp.int32)] * _NSLOT,
            [pltpu.VMEM((ch,), jnp.int32)] * (2 * _NSLOT),
            [pltpu.VMEM((ch, d), jnp.float32)] * _NSLOT,
            pltpu.VMEM_SHARED((npad, d), jnp.float32),
            [pltpu.SemaphoreType.DMA] * _NSLOT,
            [pltpu.SemaphoreType.DMA] * _NSLOT,
            [pltpu.SemaphoreType.DMA] * _NSLOT,
            [pltpu.SemaphoreType.DMA] * _NSLOT,
            pltpu.SemaphoreType.DMA,
        ],
    )
    def k(h_hbm, src_hbm, dst_hbm, z_hbm, out_hbm,
          sidx, didx, rows, agg, gsem, ssem, isem, dsem, zsem):
        c = lax.axis_index("c")
        s = lax.axis_index("s")
        wid = s * _NC + c
        base = wid * per_w
        rbase = s * rows_per_sub

        def soff(i):
            return base + jnp.minimum(i, last) * ch

        def idx_cp(i, q, dslot):
            a = pltpu.make_async_copy(src_hbm.at[pl.ds(soff(i), ch)],
                                      sidx[q], isem[q])
            b = pltpu.make_async_copy(dst_hbm.at[pl.ds(soff(i), ch)],
                                      didx[dslot], dsem[q])
            return a, b

        def gath(q):
            return pltpu.make_async_copy(h_hbm.at[sidx[q]], rows[q], gsem[q])

        def scat(q, dslot):
            return pltpu.make_async_copy(rows[q], agg.at[didx[dslot]],
                                         ssem[q])

        def zero_cp():
            return pltpu.make_async_copy(
                z_hbm.at[pl.ds(rbase, rows_per_sub)],
                agg.at[pl.ds(rbase, rows_per_sub)], zsem)

        zero_cp().start()

        def halfblock(c0, lo, hi, first):
            gs = []
            for q in range(_NSLOT):
                if not first:
                    scat(q, (hi + q) % ds2).wait()
                a, b = idx_cp(c0 + lo + q, q, lo + q)
                a.wait()
                b.wait()
                g = gath(q)
                g.start()
                gs.append(g)
            for q in range(_NSLOT):
                gs[q].wait()
                if first:
                    zero_cp().wait()
                    plsc.subcore_barrier()
                    first = False
                scat(q, lo + q).start(add=True)
                for cp in idx_cp(c0 + hi + q, q, (hi + q) % ds2):
                    cp.start()

        def block(j, first):
            c0 = ds2 * j
            if first:
                for q in range(_NSLOT):
                    for cp in idx_cp(c0 + q, q, q):
                        cp.start()
            halfblock(c0, 0, _NSLOT, first)
            halfblock(c0, _NSLOT, ds2, False)

        block(0, True)

        @pl.loop(1, nblk)
        def _(j):
            block(j, False)

        for q in range(_NSLOT):
            scat(q, _NSLOT + q).wait()
            a, b = idx_cp(0, q, q)
            a.wait()
            b.wait()

        plsc.subcore_barrier()
        pltpu.sync_copy(agg.at[pl.ds(rbase, rows_per_sub)],
                        out_hbm.at[c].at[pl.ds(rbase, rows_per_sub)])

    return k(h, src3, dst3, zeros)


def _pad_edges(src, dst, n):
    e = src.shape[0]
    nw = _NC * _NS
    per_w = -(-e // (nw * 2 * _NSLOT * _CH)) * 2 * _NSLOT * _CH
    epad = nw * per_w
    _, npad = _seg_pad(n)
    pad_ar = jnp.arange(epad - e, dtype=jnp.int32)
    trash = n + pad_ar % (npad - n)
    src_p = jnp.concatenate([src, pad_ar % n])
    dst_p = jnp.concatenate([dst, trash])
    return src_p, dst_p


def _gin_layer(h, parts, n2g_col, eps, w1, b1, g1, be1, w2, b2, gl, bel,
               pool_input):
    n, d = h.shape
    hh = w1.shape[1]
    ng = 64

    def body(*refs):
        if pool_input:
            (h_ref, p_ref, n2g_ref, w1_ref, b1_ref, g1_ref, be1_ref,
             w2_ref, b2_ref, gl_ref, bel_ref, eps_ref,
             ho_ref, po_ref, pi_ref) = refs
        else:
            (h_ref, p_ref, n2g_ref, w1_ref, b1_ref, g1_ref, be1_ref,
             w2_ref, b2_ref, gl_ref, bel_ref, eps_ref,
             ho_ref, po_ref) = refs
        hcur = h_ref[...]
        n_rows = hcur.shape[0]
        z = (1.0 + eps_ref[0]) * hcur + p_ref[0, :n_rows] + p_ref[1, :n_rows]
        t = jnp.dot(z, w1_ref[...], precision=_PREC) + b1_ref[...]
        m = jnp.mean(t, axis=0, keepdims=True)
        v = jnp.mean((t - m) ** 2, axis=0, keepdims=True)
        u = jnp.maximum(
            g1_ref[...] * (t - m) / jnp.sqrt(v + 1e-5) + be1_ref[...], 0.0)
        t2 = jnp.dot(u, w2_ref[...], precision=_PREC) + b2_ref[...]
        m2 = jnp.mean(t2, axis=0, keepdims=True)
        v2 = jnp.mean((t2 - m2) ** 2, axis=0, keepdims=True)
        hn = jnp.maximum(
            gl_ref[...] * (t2 - m2) / jnp.sqrt(v2 + 1e-5) + bel_ref[...], 0.0)
        ho_ref[...] = hn
        onehot = (n2g_ref[...] ==
                  lax.broadcasted_iota(jnp.int32, (n, ng), 1)).astype(jnp.float32)
        dn = (((0,), (0,)), ((), ()))
        po_ref[...] = lax.dot_general(onehot, hn, dn, precision=_PREC)
        if pool_input:
            pi_ref[...] = lax.dot_general(onehot, hcur, dn, precision=_PREC)

    out_shapes = [jax.ShapeDtypeStruct((n, hh), jnp.float32),
                  jax.ShapeDtypeStruct((ng, hh), jnp.float32)]
    if pool_input:
        out_shapes.append(jax.ShapeDtypeStruct((ng, d), jnp.float32))
    in_specs = [pl.BlockSpec()] * 11 + [pl.BlockSpec(memory_space=pltpu.SMEM)]
    return pl.pallas_call(
        body,
        out_shape=out_shapes,
        in_specs=in_specs,
        out_specs=[pl.BlockSpec()] * len(out_shapes),
    )(h, parts, n2g_col, w1, b1, g1, be1, w2, b2, gl, bel, eps)


def _readout(pooled, wp, bp):
    ngr, nl, _, hh = pooled.shape
    o = wp.shape[2]

    def body(p_ref, w_ref, b_ref, o_ref):
        for g in range(ngr):
            acc = jnp.zeros((64, o), jnp.float32)
            for l in range(nl):
                acc = acc + jnp.dot(p_ref[g, l], w_ref[l], precision=_PREC)
                acc = acc + b_ref[l]
            nrm = jnp.sqrt(jnp.sum(acc * acc, axis=-1, keepdims=True))
            acc = acc / jnp.maximum(nrm, 1e-5)
            o_ref[:, g * o:(g + 1) * o] = acc

    return pl.pallas_call(
        body,
        out_shape=jax.ShapeDtypeStruct((64, ngr * o), jnp.float32),
    )(pooled, wp, bp)


def kernel(feat0, edge_index0, node2graph0, feat1, edge_index1, node2graph1,
           params):
    n, d = feat0.shape
    lps = [params['layer%d' % l] for l in range(3)]

    _, npad = _seg_pad(n)
    zeros = jnp.zeros((npad, d), jnp.float32)

    def run_graph(feat, edge_index, node2graph):
        src3, dst3 = _pad_edges(edge_index[0], edge_index[1], n)
        n2g_col = node2graph.reshape(n, 1)
        pooled = []
        h = feat
        for l, p in enumerate(lps):
            parts = _edge_segment_sum(h, src3, dst3, zeros)
            eps = jnp.reshape(p['eps'], (1,)).astype(jnp.float32)
            outs = _gin_layer(
                h, parts, n2g_col, eps,
                p['W1'], p['b1'].reshape(1, -1), p['g1'].reshape(1, -1),
                p['be1'].reshape(1, -1), p['W2'], p['b2'].reshape(1, -1),
                p['gL'].reshape(1, -1), p['beL'].reshape(1, -1),
                pool_input=(l == 0))
            if l == 0:
                h, pool_next, pool_in = outs
                pooled.append(pool_in)
            else:
                h, pool_next = outs
            pooled.append(pool_next)
        return jnp.stack(pooled)

    pooled0 = run_graph(feat0, edge_index0, node2graph0)
    pooled1 = run_graph(feat1, edge_index1, node2graph1)
    pooled = jnp.stack([pooled0, pooled1])
    wp = jnp.stack([params['pred%d' % l]['W'] for l in range(4)])
    bp = jnp.stack([params['pred%d' % l]['b'].reshape(1, -1)
                    for l in range(4)])
    return _readout(pooled, wp, bp)
